# baseline (speedup 1.0000x reference)
import jax
import jax.numpy as jnp
from jax.experimental import pallas as pl

N = 1024
NBINS = 64
NMASK = 38
K = 25
RB = 8
BIG = 1.0e6


def _body(lg_ref, rit_ref, ri_ref, cit_ref, ci_ref, out_ref):
    i = pl.program_id(0)
    x = lg_ref[...]

    bin_idx = jax.lax.broadcasted_iota(jnp.int32, (RB, N, NBINS), 2)
    mask38 = bin_idx < NMASK

    mall = jnp.max(x, axis=-1, keepdims=True)
    e = jnp.exp(x - mall)
    sall = jnp.sum(e, axis=-1)
    e38 = jnp.where(mask38, e, 0.0)
    s0 = jnp.sum(e38, axis=-1)
    s1 = jnp.sum(x * e38, axis=-1)
    ent = mall[..., 0] + jnp.log(sall) - s1 / s0

    ri_rows = rit_ref[...]
    ri_all = ri_ref[...]
    ci_rows = cit_ref[...]
    ci_all = ci_ref[...]
    resi_dist = jnp.abs(ri_rows - ri_all)
    cond = (resi_dist >= 10) | (ci_rows != ci_all)
    ent = jnp.where(cond, ent, BIG)

    colio = jax.lax.broadcasted_iota(jnp.int32, (RB, N), 1)
    acc = jnp.zeros((RB, 1), jnp.float32)
    work = ent
    for _ in range(K):
        m = jnp.min(work, axis=1, keepdims=True)
        acc = acc + m
        ismin = work == m
        first = jnp.min(jnp.where(ismin, colio, N), axis=1, keepdims=True)
        work = jnp.where(colio == first, jnp.float32(jnp.inf), work)

    partial = jnp.sum(acc) * (1.0 / (K * N))

    @pl.when(i == 0)
    def _():
        out_ref[...] = jnp.zeros_like(out_ref)

    out_ref[...] += jnp.reshape(partial, (1, 1))


def kernel(distogram_logits, residue_index, asym_id):
    lg = distogram_logits.reshape(N, N, NBINS)
    ri = residue_index.reshape(1, N).astype(jnp.int32)
    ci = asym_id.reshape(1, N).astype(jnp.int32)
    rit = ri.reshape(N, 1)
    cit = ci.reshape(N, 1)

    out = pl.pallas_call(
        _body,
        grid=(N // RB,),
        in_specs=[
            pl.BlockSpec((RB, N, NBINS), lambda i: (i, 0, 0)),
            pl.BlockSpec((RB, 1), lambda i: (i, 0)),
            pl.BlockSpec((1, N), lambda i: (0, 0)),
            pl.BlockSpec((RB, 1), lambda i: (i, 0)),
            pl.BlockSpec((1, N), lambda i: (0, 0)),
        ],
        out_specs=pl.BlockSpec((1, 1), lambda i: (0, 0)),
        out_shape=jax.ShapeDtypeStruct((1, 1), jnp.float32),
    )(lg, rit, ri, cit, ci)
    return out[0, 0]


# speedup vs baseline: 2.3570x; 2.3570x over previous
import jax
import jax.numpy as jnp
from jax.experimental import pallas as pl

N = 1024
NBINS = 64
NMASK = 38
K = 25
RB = 8
TB = 256
BIG = 1.0e6


def _ent_body(lg_ref, rit_ref, rie_ref, rio_ref, cit_ref, cie_ref, cio_ref,
              out_ref):
    H = N // 2
    M = RB * H
    x = lg_ref[...].reshape(M, 2 * NBINS)

    xc = jnp.clip(x, -60.0, 60.0)
    e = jnp.exp(xc)
    xe = xc * e
    ehi = e.astype(jnp.bfloat16)
    elo = (e - ehi.astype(jnp.float32)).astype(jnp.bfloat16)
    xehi = xe.astype(jnp.bfloat16)
    xelo = (xe - xehi.astype(jnp.float32)).astype(jnp.bfloat16)
    cin = jnp.concatenate([ehi, elo, xehi, xelo], axis=1)

    row = jax.lax.broadcasted_iota(jnp.int32, (4 * 2 * NBINS, 8), 0)
    col = jax.lax.broadcasted_iota(jnp.int32, (4 * 2 * NBINS, 8), 1)
    lane = row % (2 * NBINS)
    from_e = row < 2 * 2 * NBINS
    even_all = lane < NBINS
    odd_all = lane >= NBINS
    even_38 = lane < NMASK
    odd_38 = (lane >= NBINS) & (lane < NBINS + NMASK)
    w = jnp.where(
        (col == 0) & from_e & even_all, 1.0,
        jnp.where(
            (col == 1) & from_e & odd_all, 1.0,
            jnp.where(
                (col == 2) & from_e & even_38, 1.0,
                jnp.where(
                    (col == 3) & from_e & odd_38, 1.0,
                    jnp.where(
                        (col == 4) & (~from_e) & even_38, 1.0,
                        jnp.where((col == 5) & (~from_e) & odd_38, 1.0,
                                  0.0)))))).astype(jnp.bfloat16)

    at = jax.lax.dot_general(w, cin, (((0,), (1,)), ((), ())),
                             preferred_element_type=jnp.float32)

    ent_e = (jnp.log(at[0:1, :]) - at[4:5, :] / at[2:3, :]).reshape(RB, H)
    ent_o = (jnp.log(at[1:2, :]) - at[5:6, :] / at[3:4, :]).reshape(RB, H)

    ri_rows = rit_ref[...]
    ci_rows = cit_ref[...]
    cond_e = (jnp.abs(ri_rows - rie_ref[...]) >= 10) | (ci_rows != cie_ref[...])
    cond_o = (jnp.abs(ri_rows - rio_ref[...]) >= 10) | (ci_rows != cio_ref[...])
    ent_e = jnp.where(cond_e, ent_e, BIG)
    ent_o = jnp.where(cond_o, ent_o, BIG)
    out_ref[...] = jnp.concatenate([ent_e, ent_o], axis=1)


def _topk_body(ent_ref, out_ref):
    i = pl.program_id(0)
    work = ent_ref[...]
    colio = jax.lax.broadcasted_iota(jnp.int32, (TB, N), 1)
    acc = jnp.zeros((TB, 1), jnp.float32)
    for _ in range(K):
        m = jnp.min(work, axis=1, keepdims=True)
        acc = acc + m
        ismin = work == m
        first = jnp.min(jnp.where(ismin, colio, N), axis=1, keepdims=True)
        work = jnp.where(colio == first, jnp.float32(jnp.inf), work)

    partial = jnp.sum(acc) * (1.0 / (K * N))

    @pl.when(i == 0)
    def _():
        out_ref[...] = jnp.zeros_like(out_ref)

    out_ref[...] += jnp.reshape(partial, (1, 1))


def kernel(distogram_logits, residue_index, asym_id):
    lg = distogram_logits.reshape(N, N // 2, 2 * NBINS)
    ri = residue_index.reshape(N).astype(jnp.int32)
    ci = asym_id.reshape(N).astype(jnp.int32)
    rit = ri.reshape(N, 1)
    cit = ci.reshape(N, 1)
    rie = ri[0::2].reshape(1, N // 2)
    rio = ri[1::2].reshape(1, N // 2)
    cie = ci[0::2].reshape(1, N // 2)
    cio = ci[1::2].reshape(1, N // 2)

    ent = pl.pallas_call(
        _ent_body,
        grid=(N // RB,),
        in_specs=[
            pl.BlockSpec((RB, N // 2, 2 * NBINS), lambda i: (i, 0, 0)),
            pl.BlockSpec((RB, 1), lambda i: (i, 0)),
            pl.BlockSpec((1, N // 2), lambda i: (0, 0)),
            pl.BlockSpec((1, N // 2), lambda i: (0, 0)),
            pl.BlockSpec((RB, 1), lambda i: (i, 0)),
            pl.BlockSpec((1, N // 2), lambda i: (0, 0)),
            pl.BlockSpec((1, N // 2), lambda i: (0, 0)),
        ],
        out_specs=pl.BlockSpec((RB, N), lambda i: (i, 0)),
        out_shape=jax.ShapeDtypeStruct((N, N), jnp.float32),
    )(lg, rit, rie, rio, cit, cie, cio)

    out = pl.pallas_call(
        _topk_body,
        grid=(N // TB,),
        in_specs=[pl.BlockSpec((TB, N), lambda i: (i, 0))],
        out_specs=pl.BlockSpec((1, 1), lambda i: (0, 0)),
        out_shape=jax.ShapeDtypeStruct((1, 1), jnp.float32),
    )(ent)
    return out[0, 0]


# speedup vs baseline: 3.0920x; 1.3118x over previous
import jax
import jax.numpy as jnp
from jax.experimental import pallas as pl

N = 1024
NBINS = 64
NMASK = 38
K = 25
RB = 8
TB = 256
BIG = 1.0e6


def _ent_body(lg_ref, rit_ref, ri_ref, cit_ref, ci_ref, out_ref):
    M = RB * N
    x = lg_ref[...].reshape(M, NBINS)

    xc = jnp.clip(x, -60.0, 60.0)
    e = jnp.exp(xc)
    xe = xc * e
    ehi = e.astype(jnp.bfloat16)
    elo = (e - ehi.astype(jnp.float32)).astype(jnp.bfloat16)
    xehi = xe.astype(jnp.bfloat16)
    xelo = (xe - xehi.astype(jnp.float32)).astype(jnp.bfloat16)
    cin = jnp.concatenate([ehi, elo, xehi, xelo], axis=1)

    row = jax.lax.broadcasted_iota(jnp.int32, (4 * NBINS, 8), 0)
    col = jax.lax.broadcasted_iota(jnp.int32, (4 * NBINS, 8), 1)
    lane = row % NBINS
    from_e = row < 2 * NBINS
    in38 = lane < NMASK
    w = jnp.where(
        (col == 0) & from_e, 1.0,
        jnp.where(
            (col == 1) & from_e & in38, 1.0,
            jnp.where((col == 2) & (~from_e) & in38, 1.0,
                      0.0))).astype(jnp.bfloat16)

    at = jax.lax.dot_general(w, cin, (((0,), (1,)), ((), ())),
                             preferred_element_type=jnp.float32)

    ent = (jnp.log(at[0:1, :]) - at[2:3, :] / at[1:2, :]).reshape(RB, N)

    ri_rows = rit_ref[...]
    ci_rows = cit_ref[...]
    cond = (jnp.abs(ri_rows - ri_ref[...]) >= 10) | (ci_rows != ci_ref[...])
    out_ref[...] = jnp.where(cond, ent, BIG)


def _topk_body(ent_ref, out_ref):
    i = pl.program_id(0)
    work = ent_ref[...]
    colio = jax.lax.broadcasted_iota(jnp.int32, (TB, N), 1)
    acc = jnp.zeros((TB, 1), jnp.float32)
    for _ in range(K):
        m = jnp.min(work, axis=1, keepdims=True)
        acc = acc + m
        ismin = work == m
        first = jnp.min(jnp.where(ismin, colio, N), axis=1, keepdims=True)
        work = jnp.where(colio == first, jnp.float32(jnp.inf), work)

    partial = jnp.sum(acc) * (1.0 / (K * N))

    @pl.when(i == 0)
    def _():
        out_ref[...] = jnp.zeros_like(out_ref)

    out_ref[...] += jnp.reshape(partial, (1, 1))


def kernel(distogram_logits, residue_index, asym_id):
    lg = distogram_logits.reshape(N, N, NBINS)
    ri = residue_index.reshape(1, N).astype(jnp.int32)
    ci = asym_id.reshape(1, N).astype(jnp.int32)
    rit = ri.reshape(N, 1)
    cit = ci.reshape(N, 1)

    ent = pl.pallas_call(
        _ent_body,
        grid=(N // RB,),
        in_specs=[
            pl.BlockSpec((RB, N, NBINS), lambda i: (i, 0, 0)),
            pl.BlockSpec((RB, 1), lambda i: (i, 0)),
            pl.BlockSpec((1, N), lambda i: (0, 0)),
            pl.BlockSpec((RB, 1), lambda i: (i, 0)),
            pl.BlockSpec((1, N), lambda i: (0, 0)),
        ],
        out_specs=pl.BlockSpec((RB, N), lambda i: (i, 0)),
        out_shape=jax.ShapeDtypeStruct((N, N), jnp.float32),
    )(lg, rit, ri, cit, ci)

    out = pl.pallas_call(
        _topk_body,
        grid=(N // TB,),
        in_specs=[pl.BlockSpec((TB, N), lambda i: (i, 0))],
        out_specs=pl.BlockSpec((1, 1), lambda i: (0, 0)),
        out_shape=jax.ShapeDtypeStruct((1, 1), jnp.float32),
    )(ent)
    return out[0, 0]


# speedup vs baseline: 7.9668x; 2.5766x over previous
import jax
import jax.numpy as jnp
from jax.experimental import pallas as pl

N = 1024
NBINS = 64
NMASK = 38
K = 25
RB = 8
TB = 256
BIG = 1.0e6


def _ent_body(lg_ref, rit_ref, ri_ref, cit_ref, ci_ref, out_ref):
    x = lg_ref[...]

    xc = jnp.clip(x, -60.0, 60.0)
    e = jnp.exp(xc)
    sall = jnp.sum(e, axis=1)
    s0 = jnp.sum(e[:, :NMASK, :], axis=1)
    s1 = jnp.sum(xc[:, :NMASK, :] * e[:, :NMASK, :], axis=1)

    ent = jnp.log(sall) - s1 / s0

    ri_rows = rit_ref[...]
    ci_rows = cit_ref[...]
    cond = (jnp.abs(ri_rows - ri_ref[...]) >= 10) | (ci_rows != ci_ref[...])
    out_ref[...] = jnp.where(cond, ent, BIG)


def _topk_body(ent_ref, out_ref):
    i = pl.program_id(0)
    work = ent_ref[...]
    colio = jax.lax.broadcasted_iota(jnp.int32, (TB, N), 1)
    acc = jnp.zeros((TB, 1), jnp.float32)
    for _ in range(K):
        m = jnp.min(work, axis=1, keepdims=True)
        acc = acc + m
        ismin = work == m
        first = jnp.min(jnp.where(ismin, colio, N), axis=1, keepdims=True)
        work = jnp.where(colio == first, jnp.float32(jnp.inf), work)

    partial = jnp.sum(acc) * (1.0 / (K * N))

    @pl.when(i == 0)
    def _():
        out_ref[...] = jnp.zeros_like(out_ref)

    out_ref[...] += jnp.reshape(partial, (1, 1))


def kernel(distogram_logits, residue_index, asym_id):
    lg = jnp.transpose(distogram_logits, (0, 1, 3, 2)).reshape(N, NBINS, N)
    ri = residue_index.reshape(1, N).astype(jnp.int32)
    ci = asym_id.reshape(1, N).astype(jnp.int32)
    rit = ri.reshape(N, 1)
    cit = ci.reshape(N, 1)

    ent = pl.pallas_call(
        _ent_body,
        grid=(N // RB,),
        in_specs=[
            pl.BlockSpec((RB, NBINS, N), lambda i: (i, 0, 0)),
            pl.BlockSpec((RB, 1), lambda i: (i, 0)),
            pl.BlockSpec((1, N), lambda i: (0, 0)),
            pl.BlockSpec((RB, 1), lambda i: (i, 0)),
            pl.BlockSpec((1, N), lambda i: (0, 0)),
        ],
        out_specs=pl.BlockSpec((RB, N), lambda i: (i, 0)),
        out_shape=jax.ShapeDtypeStruct((N, N), jnp.float32),
    )(lg, rit, ri, cit, ci)

    out = pl.pallas_call(
        _topk_body,
        grid=(N // TB,),
        in_specs=[pl.BlockSpec((TB, N), lambda i: (i, 0))],
        out_specs=pl.BlockSpec((1, 1), lambda i: (0, 0)),
        out_shape=jax.ShapeDtypeStruct((1, 1), jnp.float32),
    )(ent)
    return out[0, 0]
